# baseline (speedup 1.0000x reference)
import functools

import jax
import jax.numpy as jnp
from jax import lax
from jax.experimental import pallas as pl
from jax.experimental.pallas import tpu as pltpu
from jax.experimental.pallas import tpu_sc as plsc

_B = 4096
_NN = 13
_NC = 26
_NF = _NN + _NC
_D = 128
_CARDP = 1001
_RTOT = _B * _NF
_NWORK = 32
_NBUF = 6
_NREP = 32


def _sc_body(xcatf_hbm, tab_hbm, lnb_hbm, pw_hbm, pb_hbm, fid_hbm, out_hbm,
             xc_v, lnb_v, pw_v, pb_v, fid_v, nb0, nb1, *rest):
    nbufs = [nb0, nb1]
    bufs = list(rest[:_NBUF])
    gsem, wsem, nsem, xsem = (rest[_NBUF], rest[_NBUF + 1], rest[_NBUF + 2],
                              rest[_NBUF + 3])
    wid = lax.axis_index("s") * 2 + lax.axis_index("c")
    bb = wid * 128

    pltpu.async_copy(xcatf_hbm.at[:, pl.ds(bb, 128)], xc_v, xsem)
    pltpu.sync_copy(lnb_hbm, lnb_v)
    pltpu.sync_copy(pw_hbm, pw_v)
    pltpu.sync_copy(pb_hbm, pb_v)
    pltpu.sync_copy(fid_hbm, fid_v)
    pltpu.make_async_copy(xcatf_hbm.at[:, pl.ds(bb, 128)], xc_v, xsem).wait()

    for s in range(_NBUF):
        pltpu.async_copy(tab_hbm.at[s].at[xc_v.at[s]], bufs[s], gsem.at[s])

    lnb_vec = lnb_v[pl.ds(0, 16)]
    for f in range(_NN):
        pp = f % 2
        if f >= 2:
            for q in range(128 // _NREP):
                pltpu.make_async_copy(
                    nbufs[pp], out_hbm.at[pl.ds(bb, _NREP)],
                    nsem.at[pp]).wait()
        lnb_s = lnb_vec[f]
        vs = [lnb_s * pw_v[pl.ds(f * _D + j * 16, 16)]
              + pb_v[pl.ds(f * _D + j * 16, 16)]
              + fid_v[pl.ds(f * _D + j * 16, 16)] for j in range(8)]
        def rep_fn(rr, carry, pp=pp, vs=vs):
            for j in range(8):
                nbufs[pp][rr, pl.ds(j * 16, 16)] = vs[j]
            return carry
        lax.fori_loop(0, _NREP, rep_fn, 0)
        for q in range(128 // _NREP):
            pltpu.async_copy(
                nbufs[pp],
                out_hbm.at[pl.ds(f * _B + bb + q * _NREP, _NREP)],
                nsem.at[pp])
    for pp in range(2):
        for q in range(128 // _NREP):
            pltpu.make_async_copy(nbufs[pp], out_hbm.at[pl.ds(bb, _NREP)],
                                  nsem.at[pp]).wait()

    ngrp = (_NC + _NBUF - 1) // _NBUF
    for g in range(ngrp):
        for s in range(_NBUF):
            c = g * _NBUF + s
            if c >= _NC:
                break
            pltpu.make_async_copy(tab_hbm.at[c].at[xc_v.at[c]], bufs[s],
                                  gsem.at[s]).wait()
            fvs = [fid_v[pl.ds((_NN + c) * _D + j * 16, 16)]
                   for j in range(8)]
            def add_fn(rr, carry, s=s, fvs=fvs):
                for j in range(8):
                    plsc.addupdate(bufs[s].at[rr, pl.ds(j * 16, 16)], fvs[j])
                return carry
            lax.fori_loop(0, 128, add_fn, 0)
            pltpu.async_copy(
                bufs[s], out_hbm.at[pl.ds((_NN + c) * _B + bb, 128)],
                wsem.at[s])
        for s in range(_NBUF):
            c = g * _NBUF + s
            cn = c + _NBUF
            if c >= _NC:
                break
            pltpu.make_async_copy(
                bufs[s], out_hbm.at[pl.ds((_NN + c) * _B + bb, 128)],
                wsem.at[s]).wait()
            if cn < _NC:
                pltpu.async_copy(tab_hbm.at[cn].at[xc_v.at[cn]], bufs[s],
                                 gsem.at[s])


def _sc_tokenize(xcat2, cat_tables, lnb16, pwf, pbf, fidf):
    mesh = plsc.VectorSubcoreMesh(core_axis_name="c", subcore_axis_name="s")
    fn = functools.partial(
        pl.kernel,
        mesh=mesh,
        out_type=jax.ShapeDtypeStruct((_RTOT, _D), jnp.float32),
        scratch_types=(
            [pltpu.VMEM((_NC, 128), jnp.int32),
             pltpu.VMEM((16,), jnp.float32),
             pltpu.VMEM((16 * _D,), jnp.float32),
             pltpu.VMEM((16 * _D,), jnp.float32),
             pltpu.VMEM((_NF * _D,), jnp.float32),
             pltpu.VMEM((_NREP, _D), jnp.float32),
             pltpu.VMEM((_NREP, _D), jnp.float32)]
            + [pltpu.VMEM((128, _D), jnp.float32) for _ in range(_NBUF)]
            + [pltpu.SemaphoreType.DMA((_NBUF,)),
               pltpu.SemaphoreType.DMA((_NBUF,)),
               pltpu.SemaphoreType.DMA((2,)),
               pltpu.SemaphoreType.DMA]
        ),
    )(_sc_body)
    return fn(xcat2, cat_tables, lnb16, pwf, pbf, fidf)


def kernel(x_num, x_cat, ln_w, ln_b, proj_w, proj_b, cat_tables, feat_id):
    del x_num, ln_w
    f32 = jnp.float32
    xcat2 = x_cat.astype(jnp.int32).T
    lnb16 = jnp.pad(ln_b.astype(f32), (0, 3))
    pwf = jnp.pad(proj_w.astype(f32), ((0, 3), (0, 0))).reshape(16 * _D)
    pbf = jnp.pad(proj_b.astype(f32), ((0, 3), (0, 0))).reshape(16 * _D)
    fidf = feat_id.astype(f32).reshape(_NF * _D)

    out_flat = _sc_tokenize(xcat2, cat_tables.astype(f32), lnb16, pwf, pbf,
                            fidf)
    return out_flat.reshape(_NF, _B, _D).transpose(1, 0, 2)


# speedup vs baseline: 1.0158x; 1.0158x over previous
import functools

import jax
import jax.numpy as jnp
from jax import lax
from jax.experimental import pallas as pl
from jax.experimental.pallas import tpu as pltpu
from jax.experimental.pallas import tpu_sc as plsc

_B = 4096
_NN = 13
_NC = 26
_NF = _NN + _NC
_D = 128
_CARDP = 1001
_RTOT = _B * _NF
_NWORK = 32
_NBUF = 5
_NREP = 128


def _sc_body(xcatf_hbm, tab_hbm, lnb_hbm, pw_hbm, pb_hbm, fid_hbm, out_hbm,
             xc_v, lnb_v, pw_v, pb_v, fid_v, nb0, nb1, *rest):
    nbufs = [nb0, nb1]
    bufs = list(rest[:_NBUF])
    gsem, wsem, nsem, xsem = (rest[_NBUF], rest[_NBUF + 1], rest[_NBUF + 2],
                              rest[_NBUF + 3])
    wid = lax.axis_index("s") * 2 + lax.axis_index("c")
    bb = wid * 128

    pltpu.async_copy(xcatf_hbm.at[:, pl.ds(bb, 128)], xc_v, xsem)
    pltpu.sync_copy(lnb_hbm, lnb_v)
    pltpu.sync_copy(pw_hbm, pw_v)
    pltpu.sync_copy(pb_hbm, pb_v)
    pltpu.sync_copy(fid_hbm, fid_v)
    pltpu.make_async_copy(xcatf_hbm.at[:, pl.ds(bb, 128)], xc_v, xsem).wait()

    for s in range(_NBUF):
        pltpu.async_copy(tab_hbm.at[s].at[xc_v.at[s]], bufs[s], gsem.at[s])

    lnb_vec = lnb_v[pl.ds(0, 16)]
    for f in range(_NN):
        pp = f % 2
        if f >= 2:
            for q in range(128 // _NREP):
                pltpu.make_async_copy(
                    nbufs[pp], out_hbm.at[pl.ds(bb, _NREP)],
                    nsem.at[pp]).wait()
        lnb_s = lnb_vec[f]
        vs = [lnb_s * pw_v[pl.ds(f * _D + j * 16, 16)]
              + pb_v[pl.ds(f * _D + j * 16, 16)]
              + fid_v[pl.ds(f * _D + j * 16, 16)] for j in range(8)]
        def rep_fn(rr, carry, pp=pp, vs=vs):
            for j in range(8):
                nbufs[pp][rr, pl.ds(j * 16, 16)] = vs[j]
            return carry
        lax.fori_loop(0, _NREP, rep_fn, 0)
        for q in range(128 // _NREP):
            pltpu.async_copy(
                nbufs[pp],
                out_hbm.at[pl.ds(f * _B + bb + q * _NREP, _NREP)],
                nsem.at[pp])
    for pp in range(2):
        for q in range(128 // _NREP):
            pltpu.make_async_copy(nbufs[pp], out_hbm.at[pl.ds(bb, _NREP)],
                                  nsem.at[pp]).wait()

    ngrp = (_NC + _NBUF - 1) // _NBUF
    for g in range(ngrp):
        for s in range(_NBUF):
            c = g * _NBUF + s
            if c >= _NC:
                break
            pltpu.make_async_copy(tab_hbm.at[c].at[xc_v.at[c]], bufs[s],
                                  gsem.at[s]).wait()
            fvs = [fid_v[pl.ds((_NN + c) * _D + j * 16, 16)]
                   for j in range(8)]
            def add_fn(rr, carry, s=s, fvs=fvs):
                for j in range(8):
                    plsc.addupdate(bufs[s].at[rr, pl.ds(j * 16, 16)], fvs[j])
                return carry
            lax.fori_loop(0, 128, add_fn, 0)
            pltpu.async_copy(
                bufs[s], out_hbm.at[pl.ds((_NN + c) * _B + bb, 128)],
                wsem.at[s])
        for s in range(_NBUF):
            c = g * _NBUF + s
            cn = c + _NBUF
            if c >= _NC:
                break
            pltpu.make_async_copy(
                bufs[s], out_hbm.at[pl.ds((_NN + c) * _B + bb, 128)],
                wsem.at[s]).wait()
            if cn < _NC:
                pltpu.async_copy(tab_hbm.at[cn].at[xc_v.at[cn]], bufs[s],
                                 gsem.at[s])


def _sc_tokenize(xcat2, cat_tables, lnb16, pwf, pbf, fidf):
    mesh = plsc.VectorSubcoreMesh(core_axis_name="c", subcore_axis_name="s")
    fn = functools.partial(
        pl.kernel,
        mesh=mesh,
        out_type=jax.ShapeDtypeStruct((_RTOT, _D), jnp.float32),
        scratch_types=(
            [pltpu.VMEM((_NC, 128), jnp.int32),
             pltpu.VMEM((16,), jnp.float32),
             pltpu.VMEM((16 * _D,), jnp.float32),
             pltpu.VMEM((16 * _D,), jnp.float32),
             pltpu.VMEM((_NF * _D,), jnp.float32),
             pltpu.VMEM((_NREP, _D), jnp.float32),
             pltpu.VMEM((_NREP, _D), jnp.float32)]
            + [pltpu.VMEM((128, _D), jnp.float32) for _ in range(_NBUF)]
            + [pltpu.SemaphoreType.DMA((_NBUF,)),
               pltpu.SemaphoreType.DMA((_NBUF,)),
               pltpu.SemaphoreType.DMA((2,)),
               pltpu.SemaphoreType.DMA]
        ),
    )(_sc_body)
    return fn(xcat2, cat_tables, lnb16, pwf, pbf, fidf)


def kernel(x_num, x_cat, ln_w, ln_b, proj_w, proj_b, cat_tables, feat_id):
    del x_num, ln_w
    f32 = jnp.float32
    xcat2 = x_cat.astype(jnp.int32).T
    lnb16 = jnp.pad(ln_b.astype(f32), (0, 3))
    pwf = jnp.pad(proj_w.astype(f32), ((0, 3), (0, 0))).reshape(16 * _D)
    pbf = jnp.pad(proj_b.astype(f32), ((0, 3), (0, 0))).reshape(16 * _D)
    fidf = feat_id.astype(f32).reshape(_NF * _D)

    out_flat = _sc_tokenize(xcat2, cat_tables.astype(f32), lnb16, pwf, pbf,
                            fidf)
    return out_flat.reshape(_NF, _B, _D).transpose(1, 0, 2)


# speedup vs baseline: 1.0166x; 1.0008x over previous
import functools

import jax
import jax.numpy as jnp
from jax import lax
from jax.experimental import pallas as pl
from jax.experimental.pallas import tpu as pltpu
from jax.experimental.pallas import tpu_sc as plsc

_B = 4096
_NN = 13
_NC = 26
_NF = _NN + _NC
_D = 128
_RTOT = _B * _NF
_NWORK = 32
_NBUF = 5
_NREP = 128


def _sc_body(xcatf_hbm, tab_hbm, lnb_hbm, pw_hbm, pb_hbm, fid_hbm, out_hbm,
             xc_v, lnb_v, pw_v, pb_v, fid_v, nb0, nb1, *rest):
    nbufs = [nb0, nb1]
    bufs = list(rest[:_NBUF])
    gsem, wsem, nsem, xsem = (rest[_NBUF], rest[_NBUF + 1], rest[_NBUF + 2],
                              rest[_NBUF + 3])
    wid = lax.axis_index("s") * 2 + lax.axis_index("c")
    bb = wid * 128

    pltpu.async_copy(xcatf_hbm.at[:, pl.ds(bb, 128)], xc_v, xsem)
    pltpu.sync_copy(lnb_hbm, lnb_v)
    pltpu.sync_copy(pw_hbm, pw_v)
    pltpu.sync_copy(pb_hbm, pb_v)
    pltpu.sync_copy(fid_hbm, fid_v)
    pltpu.make_async_copy(xcatf_hbm.at[:, pl.ds(bb, 128)], xc_v, xsem).wait()

    for s in range(_NBUF):
        pltpu.async_copy(tab_hbm.at[s].at[xc_v.at[s]], bufs[s], gsem.at[s])

    lnb_vec = lnb_v[pl.ds(0, 16)]
    for f in range(_NN):
        pp = f % 2
        if f >= 2:
            for q in range(128 // _NREP):
                pltpu.make_async_copy(
                    nbufs[pp], out_hbm.at[pl.ds(bb, _NREP)],
                    nsem.at[pp]).wait()
        lnb_s = lnb_vec[f]
        vs = [lnb_s * pw_v[pl.ds(f * _D + j * 16, 16)]
              + pb_v[pl.ds(f * _D + j * 16, 16)]
              + fid_v[pl.ds(f * _D + j * 16, 16)] for j in range(8)]
        def rep_fn(rr, carry, pp=pp, vs=vs):
            for j in range(8):
                nbufs[pp][rr, pl.ds(j * 16, 16)] = vs[j]
            return carry
        lax.fori_loop(0, _NREP, rep_fn, 0)
        for q in range(128 // _NREP):
            pltpu.async_copy(
                nbufs[pp],
                out_hbm.at[pl.ds(f * _B + bb + q * _NREP, _NREP)],
                nsem.at[pp])
    for pp in range(2):
        for q in range(128 // _NREP):
            pltpu.make_async_copy(nbufs[pp], out_hbm.at[pl.ds(bb, _NREP)],
                                  nsem.at[pp]).wait()

    ngrp = (_NC + _NBUF - 1) // _NBUF
    for g in range(ngrp):
        for s in range(_NBUF):
            c = g * _NBUF + s
            if c >= _NC:
                break
            pltpu.make_async_copy(tab_hbm.at[c].at[xc_v.at[c]], bufs[s],
                                  gsem.at[s]).wait()
            fvs = [fid_v[pl.ds((_NN + c) * _D + j * 16, 16)]
                   for j in range(8)]
            def add_fn(rr, carry, s=s, fvs=fvs):
                for j in range(8):
                    plsc.addupdate(bufs[s].at[rr, pl.ds(j * 16, 16)], fvs[j])
                return carry
            lax.fori_loop(0, 128, add_fn, 0)
            pltpu.async_copy(
                bufs[s], out_hbm.at[pl.ds((_NN + c) * _B + bb, 128)],
                wsem.at[s])
        for s in range(_NBUF):
            c = g * _NBUF + s
            cn = c + _NBUF
            if c >= _NC:
                break
            pltpu.make_async_copy(
                bufs[s], out_hbm.at[pl.ds((_NN + c) * _B + bb, 128)],
                wsem.at[s]).wait()
            if cn < _NC:
                pltpu.async_copy(tab_hbm.at[cn].at[xc_v.at[cn]], bufs[s],
                                 gsem.at[s])


def _sc_tokenize(xcat2, cat_tables, lnb16, pwf, pbf, fidf):
    mesh = plsc.VectorSubcoreMesh(core_axis_name="c", subcore_axis_name="s")
    fn = functools.partial(
        pl.kernel,
        mesh=mesh,
        out_type=jax.ShapeDtypeStruct((_RTOT, _D), jnp.float32),
        scratch_types=(
            [pltpu.VMEM((_NC, 128), jnp.int32),
             pltpu.VMEM((16,), jnp.float32),
             pltpu.VMEM((16 * _D,), jnp.float32),
             pltpu.VMEM((16 * _D,), jnp.float32),
             pltpu.VMEM((_NF * _D,), jnp.float32),
             pltpu.VMEM((_NREP, _D), jnp.float32),
             pltpu.VMEM((_NREP, _D), jnp.float32)]
            + [pltpu.VMEM((128, _D), jnp.float32) for _ in range(_NBUF)]
            + [pltpu.SemaphoreType.DMA((_NBUF,)),
               pltpu.SemaphoreType.DMA((_NBUF,)),
               pltpu.SemaphoreType.DMA((2,)),
               pltpu.SemaphoreType.DMA]
        ),
    )(_sc_body)
    return fn(xcat2, cat_tables, lnb16, pwf, pbf, fidf)


def kernel(x_num, x_cat, ln_w, ln_b, proj_w, proj_b, cat_tables, feat_id):
    del x_num, ln_w
    f32 = jnp.float32
    xcat2 = x_cat.astype(jnp.int32).T
    lnb16 = jnp.pad(ln_b.astype(f32), (0, 3))
    pwf = jnp.pad(proj_w.astype(f32), ((0, 3), (0, 0))).reshape(16 * _D)
    pbf = jnp.pad(proj_b.astype(f32), ((0, 3), (0, 0))).reshape(16 * _D)
    fidf = feat_id.astype(f32).reshape(_NF * _D)

    out_flat = _sc_tokenize(xcat2, cat_tables.astype(f32), lnb16, pwf, pbf,
                            fidf)
    return out_flat.reshape(_NF, _B, _D).transpose(1, 0, 2)
